# baseline (speedup 1.0000x reference)
import functools

import jax
import jax.numpy as jnp
from jax import lax
from jax.experimental import pallas as pl
from jax.experimental.pallas import tpu as pltpu
from jax.experimental.pallas import tpu_sc as plsc

USERS = 1000000
DIM = 64
B = 16384

NC = 2
NS = 16
NW = NC * NS

LANES = 16
N_SCAN = B // LANES
CHUNK_U = 1024
N_CHUNKS = (USERS + CHUNK_U - 1) // CHUNK_U
K_MAX = (N_CHUNKS + NW - 1) // NW
LAST_U0 = ((USERS - CHUNK_U + 127) // 128) * 128
OUT_ROWS = B + 16


@functools.lru_cache(maxsize=1)
def _build():
  mesh = plsc.VectorSubcoreMesh(core_axis_name="c", subcore_axis_name="s")

  @functools.partial(
      pl.kernel,
      mesh=mesh,
      compiler_params=pltpu.CompilerParams(
          use_tc_tiling_on_sc=True, needs_layout_passes=False),
      out_type=jax.ShapeDtypeStruct((OUT_ROWS, 2 * DIM), jnp.float32),
      scratch_types=[
          pltpu.VMEM((B,), jnp.int32),
          pltpu.VMEM((B,), jnp.int32),
          pltpu.VMEM((B,), jnp.int32),
          pltpu.VMEM((DIM, CHUNK_U), jnp.float32),
          pltpu.VMEM((LANES, 2 * DIM), jnp.float32),
          pltpu.VMEM((LANES,), jnp.int32),
          pltpu.SemaphoreType.DMA,
          pltpu.SemaphoreType.DMA,
      ],
  )
  def gather_kernel(idx_hbm, tab_hbm, out_hbm, xv, u_list, b_list,
                    chunk_v, rows_v, bsafe_v, sem, sem_out):
    wid = lax.axis_index("s") * NC + lax.axis_index("c")
    pltpu.sync_copy(idx_hbm, xv)

    lanes = lax.iota(jnp.int32, LANES)
    wid_v = jnp.full((LANES,), wid, jnp.int32)

    def scan_grp(g, off_v):
      v_u = xv[pl.ds(g * LANES, LANES)]
      mine = ((v_u >> 10) & (NW - 1)) == wid_v
      mine_i = mine.astype(jnp.int32)
      pos = off_v + plsc.cumsum(mine_i) - mine_i
      plsc.store_scatter(u_list, [pos], v_u, mask=mine)
      plsc.store_scatter(b_list, [pos], g * LANES + lanes, mask=mine)
      return off_v + plsc.all_reduce_population_count(mine)

    off_v = lax.fori_loop(0, N_SCAN, scan_grp, jnp.zeros((LANES,), jnp.int32))
    n_pairs = jnp.max(off_v)
    n_grp = (n_pairs + LANES - 1) // LANES

    def do_chunk(k, carry):
      c = jnp.minimum(k * NW + wid, N_CHUNKS - 1)
      u0 = jnp.minimum(c * CHUNK_U, LAST_U0)
      u0 = pl.multiple_of(u0, 128)
      pltpu.async_copy(
          tab_hbm.at[:, pl.ds(u0, CHUNK_U)], chunk_v, sem).wait()
      u0_v = jnp.full((LANES,), 1, jnp.int32) * u0

      def pair_grp(m, carry2):
        v_u = u_list[pl.ds(m * LANES, LANES)]
        hit = (v_u >= u0_v) & (v_u < u0_v + CHUNK_U)
        n_hit = plsc.all_reduce_population_count(hit)

        @pl.when(jnp.max(n_hit) > 0)
        def _():
          v_b = b_list[pl.ds(m * LANES, LANES)]
          u_loc = jnp.where(hit, v_u - u0_v, 0)
          for q in range(DIM):
            q_v = jnp.full((LANES,), q, jnp.int32)
            val = plsc.load_gather(chunk_v, [q_v, u_loc])
            plsc.store_scatter(rows_v, [lanes, q_v], val)
          bsafe_v[...] = jnp.where(hit, v_b, jnp.full((LANES,), B, jnp.int32))

        return carry2

      lax.fori_loop(0, n_grp, pair_grp, 0)
      return carry

    lax.fori_loop(0, K_MAX, do_chunk, 0)

  return gather_kernel


def kernel(x, table):
  xi = x.astype(jnp.int32)
  wide = _build()(xi, table.T)
  return wide[:B, :DIM]


# speedup vs baseline: 1.2379x; 1.2379x over previous
import functools

import jax
import jax.numpy as jnp
from jax import lax
from jax.experimental import pallas as pl
from jax.experimental.pallas import tpu as pltpu
from jax.experimental.pallas import tpu_sc as plsc

USERS = 1000000
USERS_PAD = 1000064
DIM = 64
B = 16384

NC = 2
NS = 16
NW = NC * NS
LANES = 16

N_SCAN = B // LANES
CHUNK_U = 256
N_CHUNKS = (USERS + CHUNK_U - 1) // CHUNK_U
K_REAL = (N_CHUNKS + NW - 1) // NW
K_MAX = K_REAL + (K_REAL % 2)
LAST_U0 = USERS_PAD - CHUNK_U
CAP = 32
STAGE_CAP = 128
OUT_ROWS = B + 16


@functools.lru_cache(maxsize=1)
def _build():
  mesh = plsc.VectorSubcoreMesh(core_axis_name="c", subcore_axis_name="s")

  @functools.partial(
      pl.kernel,
      mesh=mesh,
      compiler_params=pltpu.CompilerParams(
          use_tc_tiling_on_sc=True, needs_layout_passes=False),
      out_type=jax.ShapeDtypeStruct((OUT_ROWS, 2 * DIM), jnp.float32),
      scratch_types=[
          pltpu.VMEM((B,), jnp.int32),
          pltpu.VMEM((B,), jnp.int32),
          pltpu.VMEM((B,), jnp.int32),
          pltpu.VMEM((K_MAX * CAP,), jnp.int32),
          pltpu.VMEM((2 * DIM,), jnp.int32),
          pltpu.VMEM((DIM, CHUNK_U), jnp.float32),
          pltpu.VMEM((DIM, CHUNK_U), jnp.float32),
          pltpu.VMEM((STAGE_CAP, 2 * DIM), jnp.float32),
          pltpu.VMEM((STAGE_CAP,), jnp.int32),
          pltpu.SemaphoreType.DMA,
          pltpu.SemaphoreType.DMA,
          pltpu.SemaphoreType.DMA,
      ],
  )
  def gather_kernel(idx_hbm, tab_hbm, out_hbm, xv, b_list, ovf_b, bkt_v,
                    cnt_v, chunk_a, chunk_b, stage_v, bstage_v,
                    sem_a, sem_b, sem_out):
    wid = lax.axis_index("s") * NC + lax.axis_index("c")
    pltpu.sync_copy(idx_hbm, xv)

    lanes = lax.iota(jnp.int32, LANES)
    one_v = jnp.full((LANES,), 1, jnp.int32)
    zero_v = jnp.zeros((LANES,), jnp.int32)
    dump_v = jnp.full((LANES,), B, jnp.int32)
    wid_v = one_v * wid

    def scan_grp(g, off_v):
      v_u = xv[pl.ds(g * LANES, LANES)]
      mine = ((v_u >> 8) & (NW - 1)) == wid_v
      mine_i = mine.astype(jnp.int32)
      pos = off_v + plsc.cumsum(mine_i) - mine_i
      plsc.store_scatter(b_list, [pos], g * LANES + lanes, mask=mine)
      return off_v + plsc.all_reduce_population_count(mine)

    np_v = lax.fori_loop(0, N_SCAN, scan_grp, zero_v)
    n_pairs = jnp.max(np_v)
    n_grp = (n_pairs + LANES - 1) // LANES

    for r in range(2 * DIM // LANES):
      cnt_v[pl.ds(r * LANES, LANES)] = zero_v

    def bucket_grp(m, ovf_off_v):
      valid = (m * LANES + lanes) < np_v
      v_b = b_list[pl.ds(m * LANES, LANES)]
      v_u = plsc.load_gather(xv, [v_b], mask=valid)
      k_of = v_u >> 13
      for qi in range(LANES):
        m_one = valid & (lanes == qi)
        cnt = plsc.load_gather(cnt_v, [k_of], mask=m_one)
        in_cap = cnt < CAP
        plsc.store_scatter(bkt_v, [k_of * CAP + cnt], v_b,
                           mask=m_one & in_cap)
        plsc.store_scatter(cnt_v, [k_of], cnt + 1, mask=m_one & in_cap)
        spill = m_one & ~in_cap
        plsc.store_scatter(ovf_b, [ovf_off_v], v_b, mask=spill)
        ovf_off_v = ovf_off_v + plsc.all_reduce_population_count(spill)
      return ovf_off_v

    ovf_v = lax.fori_loop(0, n_grp, bucket_grp, zero_v)
    n_ovf = jnp.max(ovf_v)

    for r in range(STAGE_CAP // LANES):
      bstage_v[pl.ds(r * LANES, LANES)] = dump_v

    def u0_of(k):
      c = jnp.minimum(k * NW + wid, N_CHUNKS - 1)
      return pl.multiple_of(jnp.minimum(c * CHUNK_U, LAST_U0), 128)

    def issue(k, buf, sem):
      @pl.when(k < K_REAL)
      def _():
        pltpu.async_copy(tab_hbm.at[:, pl.ds(u0_of(k), CHUNK_U)], buf, sem)

    def extract_grp(v_b, hit, u0_v, off_v, chunk):
      v_u = plsc.load_gather(xv, [v_b], mask=hit)
      hit = hit & (v_u >= u0_v) & (v_u < u0_v + CHUNK_U)
      hit_i = hit.astype(jnp.int32)
      pos = off_v + plsc.cumsum(hit_i) - hit_i
      u_loc = jnp.where(hit, v_u - u0_v, zero_v)

      def dim_grp(qq, carry):
        for qi in range(LANES):
          q1_v = one_v * (qq * LANES + qi)
          val = plsc.load_gather(chunk, [q1_v, u_loc], mask=hit)
          plsc.store_scatter(stage_v, [pos, q1_v], val, mask=hit)
        return carry

      lax.fori_loop(0, DIM // LANES, dim_grp, 0)
      plsc.store_scatter(bstage_v, [pos], v_b, mask=hit)
      off2 = off_v + plsc.all_reduce_population_count(hit)
      need_flush = jnp.max(off2) > STAGE_CAP - LANES

      @pl.when(need_flush)
      def _():
        pltpu.async_copy(stage_v, out_hbm.at[bstage_v], sem_out).wait()
        for r in range(STAGE_CAP // LANES):
          bstage_v[pl.ds(r * LANES, LANES)] = dump_v

      return jnp.where(need_flush, zero_v, off2)

    def process(k, off_v, chunk, sem):
      u0 = u0_of(k)

      @pl.when(k < K_REAL)
      def _():
        pltpu.make_async_copy(
            tab_hbm.at[:, pl.ds(u0, CHUNK_U)], chunk, sem).wait()
      u0_v = one_v * u0
      cnt_k = plsc.load_gather(cnt_v, [one_v * k])
      base = k * CAP
      for g in range(CAP // LANES):
        sub = lanes + g * LANES
        hit = sub < jnp.minimum(cnt_k, CAP)
        has = jnp.max(jnp.where(hit, one_v, zero_v)) > 0

        def body(off_in, hit=hit, g=g):
          v_b = bkt_v[pl.ds(base + g * LANES, LANES)]
          return extract_grp(v_b, hit, u0_v, off_in, chunk)

        off_v = _when_carry(has, body, off_v)
      return off_v

    issue(jnp.int32(0), chunk_a, sem_a)
    issue(jnp.int32(1), chunk_b, sem_b)

    def kk_body(kk, off_v):
      k0 = kk * 2
      off_v = process(k0, off_v, chunk_a, sem_a)
      issue(k0 + 2, chunk_a, sem_a)
      off_v = process(k0 + 1, off_v, chunk_b, sem_b)
      issue(k0 + 3, chunk_b, sem_b)
      return off_v

    off_v = lax.fori_loop(0, K_MAX // 2, kk_body, zero_v)

    @pl.when(jnp.max(off_v) > 0)
    def _():
      pltpu.async_copy(stage_v, out_hbm.at[bstage_v], sem_out).wait()
      for r in range(STAGE_CAP // LANES):
        bstage_v[pl.ds(r * LANES, LANES)] = dump_v

    @pl.when(n_ovf > 0)
    def _():
      n_og = (n_ovf + LANES - 1) // LANES

      def ovf_chunk(k, off_v):
        u0 = u0_of(k)
        pltpu.async_copy(
            tab_hbm.at[:, pl.ds(u0, CHUNK_U)], chunk_a, sem_a).wait()
        u0_v = one_v * u0

        def ovf_grp(m, off_in):
          valid = (m * LANES + lanes) < ovf_v
          v_b = ovf_b[pl.ds(m * LANES, LANES)]
          return extract_grp(v_b, valid, u0_v, off_in, chunk_a)

        return lax.fori_loop(0, n_og, ovf_grp, off_v)

      off3 = lax.fori_loop(0, K_REAL, ovf_chunk, zero_v)

      @pl.when(jnp.max(off3) > 0)
      def _():
        pltpu.async_copy(stage_v, out_hbm.at[bstage_v], sem_out).wait()

  return gather_kernel


def _when_carry(cond, body, carry):
  return lax.cond(cond, body, lambda c: c, carry)


def kernel(x, table):
  xi = x.astype(jnp.int32)
  wide = _build()(xi, table.T)
  return wide[:B, :DIM]


# speedup vs baseline: 1.3866x; 1.1202x over previous
import functools

import jax
import jax.numpy as jnp
from jax import lax
from jax.experimental import pallas as pl
from jax.experimental.pallas import tpu as pltpu
from jax.experimental.pallas import tpu_sc as plsc

USERS = 1000000
USERS_PAD = 1000064
DIM = 64
B = 16384

NC = 2
NS = 16
NW = NC * NS
LANES = 16

N_SCAN = B // LANES
CHUNK_U = 512
CHUNK_SH = 9
N_CHUNKS = (USERS + CHUNK_U - 1) // CHUNK_U
K_REAL = (N_CHUNKS + NW - 1) // NW
K_MAX = K_REAL + (K_REAL % 2)
LAST_U0 = USERS_PAD - CHUNK_U
CAP = 24
STAGE_CAP = 112
OUT_ROWS = B + 16


@functools.lru_cache(maxsize=1)
def _build():
  mesh = plsc.VectorSubcoreMesh(core_axis_name="c", subcore_axis_name="s")

  @functools.partial(
      pl.kernel,
      mesh=mesh,
      compiler_params=pltpu.CompilerParams(
          use_tc_tiling_on_sc=True, needs_layout_passes=False),
      out_type=jax.ShapeDtypeStruct((OUT_ROWS, 2 * DIM), jnp.float32),
      scratch_types=[
          pltpu.VMEM((B,), jnp.int32),
          pltpu.VMEM((B,), jnp.int32),
          pltpu.VMEM((B,), jnp.int32),
          pltpu.VMEM((K_MAX * CAP + LANES,), jnp.int32),
          pltpu.VMEM((DIM,), jnp.int32),
          pltpu.VMEM((DIM, CHUNK_U), jnp.float32),
          pltpu.VMEM((DIM, CHUNK_U), jnp.float32),
          pltpu.VMEM((STAGE_CAP, 2 * DIM), jnp.float32),
          pltpu.VMEM((STAGE_CAP,), jnp.int32),
          pltpu.SemaphoreType.DMA,
          pltpu.SemaphoreType.DMA,
          pltpu.SemaphoreType.DMA,
      ],
  )
  def gather_kernel(idx_hbm, tab_hbm, out_hbm, xv, b_list, ovf_b, bkt_v,
                    cnt_v, chunk_a, chunk_b, stage_v, bstage_v,
                    sem_a, sem_b, sem_out):
    wid = lax.axis_index("s") * NC + lax.axis_index("c")
    pltpu.sync_copy(idx_hbm, xv)

    lanes = lax.iota(jnp.int32, LANES)
    one_v = jnp.full((LANES,), 1, jnp.int32)
    zero_v = jnp.zeros((LANES,), jnp.int32)
    dump_v = jnp.full((LANES,), B, jnp.int32)
    wid_v = one_v * wid

    def u0_of(k):
      c = jnp.minimum(k * NW + wid, N_CHUNKS - 1)
      return pl.multiple_of(jnp.minimum(c * CHUNK_U, LAST_U0), 128)

    def issue(k, buf, sem):
      @pl.when(k < K_REAL)
      def _():
        pltpu.async_copy(tab_hbm.at[:, pl.ds(u0_of(k), CHUNK_U)], buf, sem)

    issue(jnp.int32(0), chunk_a, sem_a)
    issue(jnp.int32(1), chunk_b, sem_b)

    def scan_grp(g, off_v):
      v_u = xv[pl.ds(g * LANES, LANES)]
      mine = ((v_u >> CHUNK_SH) & (NW - 1)) == wid_v
      mine_i = mine.astype(jnp.int32)
      pos = off_v + plsc.cumsum(mine_i) - mine_i
      plsc.store_scatter(b_list, [pos], g * LANES + lanes, mask=mine)
      return off_v + plsc.all_reduce_population_count(mine)

    np_v = lax.fori_loop(0, N_SCAN, scan_grp, zero_v)
    n_pairs = jnp.max(np_v)
    n_grp = (n_pairs + LANES - 1) // LANES

    for r in range(DIM // LANES):
      cnt_v[pl.ds(r * LANES, LANES)] = zero_v

    def bucket_grp(m, ovf_off_v):
      valid = (m * LANES + lanes) < np_v
      v_b = b_list[pl.ds(m * LANES, LANES)]
      v_u = plsc.load_gather(xv, [v_b], mask=valid)
      k_of = v_u >> (CHUNK_SH + 5)
      for qi in range(LANES):
        m_one = valid & (lanes == qi)
        cnt = plsc.load_gather(cnt_v, [k_of], mask=m_one)
        in_cap = cnt < CAP
        plsc.store_scatter(bkt_v, [k_of * CAP + cnt], v_b,
                           mask=m_one & in_cap)
        plsc.store_scatter(cnt_v, [k_of], cnt + 1, mask=m_one & in_cap)
        spill = m_one & ~in_cap
        plsc.store_scatter(ovf_b, [ovf_off_v], v_b, mask=spill)
        ovf_off_v = ovf_off_v + plsc.all_reduce_population_count(spill)
      return ovf_off_v

    ovf_v = lax.fori_loop(0, n_grp, bucket_grp, zero_v)
    n_ovf = jnp.max(ovf_v)

    for r in range(STAGE_CAP // LANES):
      bstage_v[pl.ds(r * LANES, LANES)] = dump_v

    def extract_grp(v_b, hit, u0_v, off_v, chunk):
      v_u = plsc.load_gather(xv, [v_b], mask=hit)
      hit = hit & (v_u >= u0_v) & (v_u < u0_v + CHUNK_U)
      hit_i = hit.astype(jnp.int32)
      pos = off_v + plsc.cumsum(hit_i) - hit_i
      u_loc = jnp.where(hit, v_u - u0_v, zero_v)
      for q in range(DIM):
        q1_v = one_v * q
        val = plsc.load_gather(chunk, [q1_v, u_loc], mask=hit)
        plsc.store_scatter(stage_v, [pos, q1_v], val, mask=hit)
      plsc.store_scatter(bstage_v, [pos], v_b, mask=hit)
      off2 = off_v + plsc.all_reduce_population_count(hit)
      need_flush = jnp.max(off2) > STAGE_CAP - LANES

      @pl.when(need_flush)
      def _():
        pltpu.async_copy(stage_v, out_hbm.at[bstage_v], sem_out).wait()
        for r in range(STAGE_CAP // LANES):
          bstage_v[pl.ds(r * LANES, LANES)] = dump_v

      return jnp.where(need_flush, zero_v, off2)

    def process(k, off_v, chunk, sem):
      u0 = u0_of(k)

      @pl.when(k < K_REAL)
      def _():
        pltpu.make_async_copy(
            tab_hbm.at[:, pl.ds(u0, CHUNK_U)], chunk, sem).wait()

      u0_v = one_v * u0
      cnt_k = plsc.load_gather(cnt_v, [one_v * k])
      base = k * CAP
      for g in range(2):
        sub = lanes + g * LANES
        hit = sub < jnp.minimum(cnt_k, jnp.full((LANES,), CAP, jnp.int32))
        has = jnp.max(jnp.where(hit, one_v, zero_v)) > 0

        def body(off_in, hit=hit, g=g):
          v_b = bkt_v[pl.ds(base + g * LANES, LANES)]
          return extract_grp(v_b, hit, u0_v, off_in, chunk)

        off_v = lax.cond(has, body, lambda c: c, off_v)
      return off_v

    def kk_body(kk, off_v):
      k0 = kk * 2
      off_v = process(k0, off_v, chunk_a, sem_a)
      issue(k0 + 2, chunk_a, sem_a)
      off_v = process(k0 + 1, off_v, chunk_b, sem_b)
      issue(k0 + 3, chunk_b, sem_b)
      return off_v

    off_v = lax.fori_loop(0, K_MAX // 2, kk_body, zero_v)

    @pl.when(jnp.max(off_v) > 0)
    def _():
      pltpu.async_copy(stage_v, out_hbm.at[bstage_v], sem_out).wait()
      for r in range(STAGE_CAP // LANES):
        bstage_v[pl.ds(r * LANES, LANES)] = dump_v

    @pl.when(n_ovf > 0)
    def _():
      n_og = (n_ovf + LANES - 1) // LANES

      def ovf_chunk(k, off_v):
        u0 = u0_of(k)
        pltpu.async_copy(
            tab_hbm.at[:, pl.ds(u0, CHUNK_U)], chunk_a, sem_a).wait()
        u0_v = one_v * u0

        def ovf_grp(m, off_in):
          valid = (m * LANES + lanes) < ovf_v
          v_b = ovf_b[pl.ds(m * LANES, LANES)]
          return extract_grp(v_b, valid, u0_v, off_in, chunk_a)

        return lax.fori_loop(0, n_og, ovf_grp, off_v)

      off3 = lax.fori_loop(0, K_REAL, ovf_chunk, zero_v)

      @pl.when(jnp.max(off3) > 0)
      def _():
        pltpu.async_copy(stage_v, out_hbm.at[bstage_v], sem_out).wait()

  return gather_kernel


def kernel(x, table):
  xi = x.astype(jnp.int32)
  wide = _build()(xi, table.T)
  return wide[:B, :DIM]
